# baseline (speedup 1.0000x reference)
import jax
import jax.numpy as jnp
from jax import lax
from jax.experimental import pallas as pl
from jax.experimental.pallas import tpu as pltpu

_S0, _S1, _D = 64, 64, 3
_BLOCK_B = 256
_TINY = 1e-30


def _dist_kernel(x_ref, g_ref, o_ref):
    base = pl.program_id(0) * _BLOCK_B
    g0 = g_ref[0]
    g1 = g_ref[1]
    g2 = g_ref[2]

    def body(t, carry):
        i0 = (base + t) * _D
        d0 = g0 - x_ref[i0]
        d1 = g1 - x_ref[i0 + 1]
        d2 = g2 - x_ref[i0 + 2]
        s = d0 * d0 + d1 * d1 + d2 * d2
        o_ref[t] = (s * jax.lax.rsqrt(jnp.maximum(s, _TINY))).astype(jnp.bfloat16)
        return carry

    lax.fori_loop(0, _BLOCK_B, body, None, unroll=8)


def kernel(x, grid):
    b = x.shape[0]
    h, w = _S0 // 2, _S1 * 2
    g = jnp.transpose(grid, (2, 0, 1)).reshape(_D, h, w)
    grid_spec = pltpu.PrefetchScalarGridSpec(
        num_scalar_prefetch=1,
        grid=(b // _BLOCK_B,),
        in_specs=[
            pl.BlockSpec((_D, h, w), lambda i, xp: (0, 0, 0)),
        ],
        out_specs=pl.BlockSpec((_BLOCK_B, h, w), lambda i, xp: (i, 0, 0)),
    )
    out = pl.pallas_call(
        _dist_kernel,
        grid_spec=grid_spec,
        out_shape=jax.ShapeDtypeStruct((b, h, w), jnp.bfloat16),
    )(x.reshape(b * _D), g)
    return out.astype(jnp.float32).reshape(b, _S0, _S1)
